# Initial kernel scaffold; baseline (speedup 1.0000x reference)
import jax
import jax.numpy as jnp
from jax import lax
from jax.experimental import pallas as pl
from jax.experimental.pallas import tpu as pltpu
from jax.experimental.pallas import tpu_sc as plsc

D = 128
H = 2
R = 3
ACC_W = 144
EPS = 1e-10


def _stage_a_body(x_ref, w_ref, q_ref, kv_ref):
    big = jnp.dot(x_ref[...], w_ref[...].T, preferred_element_type=jnp.float32)
    q_ref[0] = big[:, 0:128]
    kv_ref[0] = big[:, 128:896]
    q_ref[1] = big[:, 896:1024]
    kv_ref[1] = big[:, 1024:1792]


def _stage_a(x, wbig, n, bn):
    return pl.pallas_call(
        _stage_a_body,
        grid=(n // bn,),
        in_specs=[
            pl.BlockSpec((bn, D), lambda i: (i, 0)),
            pl.BlockSpec((H * (1 + 2 * R) * D, D), lambda i: (0, 0)),
        ],
        out_specs=[
            pl.BlockSpec((H, bn, D), lambda i: (0, i, 0)),
            pl.BlockSpec((H, bn, R * 2 * D), lambda i: (0, i, 0)),
        ],
        out_shape=[
            jax.ShapeDtypeStruct((H, n, D), jnp.float32),
            jax.ShapeDtypeStruct((H, n, R * 2 * D), jnp.float32),
        ],
    )(x, wbig)


def _make_sc_kernel(n, e):
    info = plsc.get_sparse_core_info()
    ncores, nsub = info.num_cores, info.num_subcores
    assert ncores == H
    eb = e // nsub
    sb = 80
    assert eb % sb == 0
    nblocks = eb // sb
    npart = n // nsub

    mesh = plsc.VectorSubcoreMesh(core_axis_name="c", subcore_axis_name="s")

    def body(q_hbm, kv_hbm, src_hbm, dst_hbm, ty_hbm, zero_hbm, out_hbm,
             src_v, dst_v, ty_v, qidx, kvidx, sidx, q_rows, kv_rows,
             sc_rows, acc, sem1, sem2):
        c = lax.axis_index("c")
        s = lax.axis_index("s")
        lane = lax.iota(jnp.int32, 16)
        pltpu.sync_copy(zero_hbm.at[pl.ds(s * npart, npart)],
                        acc.at[pl.ds(s * npart, npart)])
        ebase = s * eb
        pltpu.sync_copy(src_hbm.at[pl.ds(ebase, eb)], src_v)
        pltpu.sync_copy(dst_hbm.at[pl.ds(ebase, eb)], dst_v)
        pltpu.sync_copy(ty_hbm.at[pl.ds(ebase, eb)], ty_v)
        plsc.subcore_barrier()
        qbase = c * n
        kvbase = c * (R * n)

        def block(b, carry):
            off = b * sb
            for j in range(sb // 16):
                sl = pl.ds(off + j * 16, 16)
                jj = pl.ds(j * 16, 16)
                srcs = src_v[sl]
                dsts = dst_v[sl]
                tys = ty_v[sl]
                kvidx[jj] = srcs * R + tys + kvbase
                qidx[jj] = dsts + qbase
                sidx[jj] = dsts
            g1 = pltpu.async_copy(kv_hbm.at[kvidx], kv_rows, sem1)
            g2 = pltpu.async_copy(q_hbm.at[qidx], q_rows, sem2)
            g1.wait()
            g2.wait()

            def edge(ei, carry2):
                acc_v = q_rows[ei, pl.ds(0, 16)] * kv_rows[ei, pl.ds(0, 16)]
                for k in range(1, 8):
                    acc_v = acc_v + (q_rows[ei, pl.ds(16 * k, 16)]
                                     * kv_rows[ei, pl.ds(16 * k, 16)])
                sv = jnp.sum(acc_v)
                rv = jnp.maximum(sv, 0.0)
                nv = rv * rv * (1.0 / 256.0) + EPS
                for k in range(8):
                    sc_rows[ei, pl.ds(16 * k, 16)] = (
                        nv * kv_rows[ei, pl.ds(128 + 16 * k, 16)])
                sc_rows[ei, pl.ds(128, 16)] = jnp.where(lane == 0, nv, 0.0)
                return carry2

            lax.fori_loop(0, sb, edge, 0)
            pltpu.sync_copy(sc_rows, acc.at[sidx], add=True)
            return carry

        lax.fori_loop(0, nblocks, block, 0)
        plsc.subcore_barrier()
        pltpu.sync_copy(acc.at[pl.ds(s * npart, npart)],
                        out_hbm.at[c, pl.ds(s * npart, npart)])

    return pl.kernel(
        body,
        out_type=jax.ShapeDtypeStruct((H, n, ACC_W), jnp.float32),
        mesh=mesh,
        scratch_types=[
            pltpu.VMEM((eb,), jnp.int32),
            pltpu.VMEM((eb,), jnp.int32),
            pltpu.VMEM((eb,), jnp.int32),
            pltpu.VMEM((sb,), jnp.int32),
            pltpu.VMEM((sb,), jnp.int32),
            pltpu.VMEM((sb,), jnp.int32),
            pltpu.VMEM((sb, D), jnp.float32),
            pltpu.VMEM((sb, 2 * D), jnp.float32),
            pltpu.VMEM((sb, ACC_W), jnp.float32),
            pltpu.VMEM_SHARED((n, ACC_W), jnp.float32),
            pltpu.SemaphoreType.DMA,
            pltpu.SemaphoreType.DMA,
        ],
    )


def _stage_c_body(z_ref, wo_ref, o_ref):
    z0 = z_ref[0]
    z1 = z_ref[1]
    n0 = z0[:, 128:129]
    n1 = z1[:, 128:129]
    zn0 = z0[:, 0:128] / jnp.where(n0 > 0, n0, 1.0)
    zn1 = z1[:, 0:128] / jnp.where(n1 > 0, n1, 1.0)
    wo = wo_ref[...]
    o_ref[...] = (
        jnp.dot(zn0, wo[:, 0:128].T, preferred_element_type=jnp.float32)
        + jnp.dot(zn1, wo[:, 128:256].T, preferred_element_type=jnp.float32))


def _stage_c(z, wo, n, bn):
    return pl.pallas_call(
        _stage_c_body,
        grid=(n // bn,),
        in_specs=[
            pl.BlockSpec((H, bn, ACC_W), lambda i: (0, i, 0)),
            pl.BlockSpec((D, H * D), lambda i: (0, 0)),
        ],
        out_specs=pl.BlockSpec((bn, D), lambda i: (i, 0)),
        out_shape=jax.ShapeDtypeStruct((n, D), jnp.float32),
    )(z, wo)


@jax.jit
def kernel(node_feature, edge_index, edge_type, WQ, WK, WV, WO):
    n, d = node_feature.shape
    e = edge_index.shape[1]
    assert d == D

    parts = []
    for c in range(H):
        parts.append(WQ[c * D:(c + 1) * D])
        for r in range(R):
            parts.append(WK[r, c * D:(c + 1) * D])
            parts.append(WV[r, c * D:(c + 1) * D])
    wbig = jnp.concatenate(parts, axis=0)

    q_out, kv_out = _stage_a(node_feature, wbig, n, 400)
    q_tab = q_out.reshape(H * n, D)
    kv_tab = kv_out.reshape(H * R * n, 2 * D)

    src = edge_index[0].astype(jnp.int32)
    dst = edge_index[1].astype(jnp.int32)
    ty = edge_type.astype(jnp.int32)
    zeros = jnp.zeros((n, ACC_W), jnp.float32)

    sc = _make_sc_kernel(n, e)
    z = sc(q_tab, kv_tab, src, dst, ty, zeros)

    return _stage_c(z, WO, n, 400)


# speedup vs baseline: 32.3738x; 32.3738x over previous
import jax
import jax.numpy as jnp
from jax import lax
from jax.experimental import pallas as pl
from jax.experimental.pallas import tpu as pltpu
from jax.experimental.pallas import tpu_sc as plsc

_GDN = lax.GatherDimensionNumbers(
    offset_dims=(), collapsed_slice_dims=(0,), start_index_map=(0,))


def _vpermute(v, idx):
    return lax.gather(v, idx[:, None], dimension_numbers=_GDN,
                      slice_sizes=(1,),
                      mode=lax.GatherScatterMode.PROMISE_IN_BOUNDS)


D = 128
H = 2
R = 3
EPS = 1e-10


def _stage_a_body(x_ref, w_ref, q_ref, k_ref, v_ref):
    big = jnp.dot(x_ref[...], w_ref[...].T, preferred_element_type=jnp.float32)
    q_ref[0] = big[:, 0:128]
    q_ref[1] = big[:, 128:256]
    k_ref[0] = big[:, 256:640]
    k_ref[1] = big[:, 640:1024]
    v_ref[0] = big[:, 1024:1408]
    v_ref[1] = big[:, 1408:1792]


def _stage_a(x, wbig, n, bn):
    return pl.pallas_call(
        _stage_a_body,
        grid=(n // bn,),
        in_specs=[
            pl.BlockSpec((bn, D), lambda i: (i, 0)),
            pl.BlockSpec((H * (1 + 2 * R) * D, D), lambda i: (0, 0)),
        ],
        out_specs=[
            pl.BlockSpec((H, bn, D), lambda i: (0, i, 0)),
            pl.BlockSpec((H, bn, R * D), lambda i: (0, i, 0)),
            pl.BlockSpec((H, bn, R * D), lambda i: (0, i, 0)),
        ],
        out_shape=[
            jax.ShapeDtypeStruct((H, n, D), jnp.float32),
            jax.ShapeDtypeStruct((H, n, R * D), jnp.float32),
            jax.ShapeDtypeStruct((H, n, R * D), jnp.float32),
        ],
    )(x, wbig)


def _make_sc_kernel(n, npad, e):
    nsub = 16
    eb = e // nsub
    sb = 80
    assert eb % sb == 0
    nblocks = eb // sb
    npart = npad // nsub
    assert npart % 8 == 0
    dnr = npad // 128
    drain_rows = 32
    ndrain = npart // drain_rows

    mesh = plsc.VectorSubcoreMesh(core_axis_name="c", subcore_axis_name="s")

    def body(q_hbm, k_hbm, v_hbm, src_hbm, dst_hbm, ty_hbm, zero_hbm, out_hbm,
             src_v, dst_v, ty_v, qidx, kvidx, sidx, idbuf, q_rows, k_rows,
             v_rows, den_l, tmp, acc, den_sh, sem1, sem2, sem3):
        c = lax.axis_index("c")
        s = lax.axis_index("s")
        lane = lax.iota(jnp.int32, 16)
        pltpu.sync_copy(zero_hbm.at[pl.ds(s * npart, npart)],
                        acc.at[pl.ds(s * npart, npart)])
        pltpu.sync_copy(zero_hbm.at[pl.ds(0, dnr)], den_l)

        @pl.when(s == 0)
        def _():
            pltpu.sync_copy(zero_hbm.at[pl.ds(0, dnr)], den_sh)

        for g in range(dnr // 16):
            idbuf[pl.ds(16 * g, 16)] = lane + 16 * g
        plsc.subcore_barrier()
        ebase = s * eb
        qbase = c * n
        kvbase = c * (R * n)

        def block(b, carry):
            off = ebase + b * sb
            pltpu.sync_copy(src_hbm.at[pl.ds(off, sb)], src_v)
            pltpu.sync_copy(dst_hbm.at[pl.ds(off, sb)], dst_v)
            pltpu.sync_copy(ty_hbm.at[pl.ds(off, sb)], ty_v)
            for j in range(sb // 16):
                jj = pl.ds(j * 16, 16)
                srcs = src_v[jj]
                dsts = dst_v[jj]
                tys = ty_v[jj]
                kvidx[jj] = srcs * R + tys + kvbase
                qidx[jj] = dsts + qbase
                sidx[jj] = dsts
            g1 = pltpu.async_copy(k_hbm.at[kvidx], k_rows, sem1)
            g2 = pltpu.async_copy(q_hbm.at[qidx], q_rows, sem2)
            g3 = pltpu.async_copy(v_hbm.at[kvidx], v_rows, sem3)
            g1.wait()
            g2.wait()
            g3.wait()

            def group(g, carry2):
                base = g * 16
                numers = jnp.zeros((16,), jnp.float32)
                for j in range(16):
                    ei = base + j
                    acc_v = q_rows[ei, pl.ds(0, 16)] * k_rows[ei, pl.ds(0, 16)]
                    for k in range(1, 8):
                        acc_v = acc_v + (q_rows[ei, pl.ds(16 * k, 16)]
                                         * k_rows[ei, pl.ds(16 * k, 16)])
                    for k in (1, 2, 4, 8):
                        acc_v = acc_v + _vpermute(acc_v, lane ^ k)
                    rv = jnp.maximum(acc_v, 0.0)
                    nv = rv * rv * (1.0 / 256.0) + EPS
                    numers = numers + jnp.where(lane == j, nv, 0.0)
                    for k in range(8):
                        kk = pl.ds(16 * k, 16)
                        v_rows[ei, kk] = nv * v_rows[ei, kk]
                dsts = sidx[pl.ds(base, 16)]
                plsc.addupdate_scatter(
                    den_l, [lax.shift_right_logical(dsts, 7),
                            jnp.bitwise_and(dsts, 127)], numers)
                return carry2

            lax.fori_loop(0, sb // 16, group, 0)
            pltpu.sync_copy(v_rows, acc.at[sidx], add=True)
            return carry

        lax.fori_loop(0, nblocks, block, 0)
        plsc.subcore_barrier()
        pltpu.sync_copy(den_l, den_sh.at[idbuf], add=True)
        plsc.subcore_barrier()
        pltpu.sync_copy(den_sh, den_l)

        def drain(ch, carry):
            n0 = s * npart + ch * drain_rows
            pltpu.sync_copy(acc.at[pl.ds(n0, drain_rows)], tmp)

            def row(r, carry2):
                node = n0 + r
                dv = plsc.load_gather(
                    den_l, [jnp.full((16,), lax.shift_right_logical(node, 7),
                                     jnp.int32),
                            jnp.full((16,), jnp.bitwise_and(node, 127),
                                     jnp.int32)])
                inv = 1.0 / jnp.where(dv > 0.0, dv, 1.0)
                for k in range(8):
                    kk = pl.ds(16 * k, 16)
                    tmp[r, kk] = tmp[r, kk] * inv
                return carry2

            lax.fori_loop(0, drain_rows, row, 0)
            pltpu.sync_copy(tmp, out_hbm.at[c, pl.ds(n0, drain_rows)])
            return carry

        lax.fori_loop(0, ndrain, drain, 0)

    return pl.kernel(
        body,
        out_type=jax.ShapeDtypeStruct((H, npad, D), jnp.float32),
        mesh=mesh,
        compiler_params=pltpu.CompilerParams(needs_layout_passes=False),
        scratch_types=[
            pltpu.VMEM((sb,), jnp.int32),
            pltpu.VMEM((sb,), jnp.int32),
            pltpu.VMEM((sb,), jnp.int32),
            pltpu.VMEM((sb,), jnp.int32),
            pltpu.VMEM((sb,), jnp.int32),
            pltpu.VMEM((sb,), jnp.int32),
            pltpu.VMEM((npad // 128,), jnp.int32),
            pltpu.VMEM((sb, D), jnp.float32),
            pltpu.VMEM((sb, D), jnp.float32),
            pltpu.VMEM((sb, D), jnp.float32),
            pltpu.VMEM((npad // 128, 128), jnp.float32),
            pltpu.VMEM((32, D), jnp.float32),
            pltpu.VMEM_SHARED((npad, D), jnp.float32),
            pltpu.VMEM_SHARED((npad // 128, 128), jnp.float32),
            pltpu.SemaphoreType.DMA,
            pltpu.SemaphoreType.DMA,
            pltpu.SemaphoreType.DMA,
        ],
    )


def _stage_c_body(z_ref, wo_ref, o_ref):
    wo = wo_ref[...]
    o_ref[...] = (
        jnp.dot(z_ref[0], wo[:, 0:128].T, preferred_element_type=jnp.float32)
        + jnp.dot(z_ref[1], wo[:, 128:256].T, preferred_element_type=jnp.float32))


def _stage_c(z, wo, n, bn):
    return pl.pallas_call(
        _stage_c_body,
        grid=((n + bn - 1) // bn,),
        in_specs=[
            pl.BlockSpec((H, bn, D), lambda i: (0, i, 0)),
            pl.BlockSpec((D, H * D), lambda i: (0, 0)),
        ],
        out_specs=pl.BlockSpec((bn, D), lambda i: (i, 0)),
        out_shape=jax.ShapeDtypeStruct((n, D), jnp.float32),
    )(z, wo)


@jax.jit
def kernel(node_feature, edge_index, edge_type, WQ, WK, WV, WO):
    n, d = node_feature.shape
    e = edge_index.shape[1]
    assert d == D

    parts = [WQ[0:D], WQ[D:2 * D]]
    for c in range(H):
        for r in range(R):
            parts.append(WK[r, c * D:(c + 1) * D])
    for c in range(H):
        for r in range(R):
            parts.append(WV[r, c * D:(c + 1) * D])
    wbig = jnp.concatenate(parts, axis=0)

    q_out, k_out, v_out = _stage_a(node_feature, wbig, n, 400)
    q_tab = q_out.reshape(H * n, D)
    k_tab = k_out.reshape(H * R * n, D)
    v_tab = v_out.reshape(H * R * n, D)

    src = edge_index[0].astype(jnp.int32)
    dst = edge_index[1].astype(jnp.int32)
    ty = edge_type.astype(jnp.int32)
    npad = 10240
    zeros = jnp.zeros((npad, D), jnp.float32)

    sc = _make_sc_kernel(n, npad, e)
    z = sc(q_tab, k_tab, v_tab, src, dst, ty, zeros)

    return _stage_c(z, WO, n, 512)
